# baseline (speedup 1.0000x reference)
import functools

import jax
import jax.numpy as jnp
import numpy as np
from jax import lax
from jax.experimental import pallas as pl
from jax.experimental.pallas import tpu as pltpu
from jax.experimental.pallas import tpu_sc as plsc

DATASET_LEN = 100000
N_JOINTS = 24
N = 16384
NUM_WORKERS = 32
BPW = N // NUM_WORKERS
OUT_PW = BPW * 18
NBLK = OUT_PW // 144


def _sampler_body(table_hbm, ids_hbm, cst_hbm, out_hbm,
                  idx_v, buf, out_v, cst, sem):
    wid = lax.axis_index("s") * 2 + lax.axis_index("c")
    base = wid * BPW

    pltpu.sync_copy(cst_hbm, cst)
    for g in range(3):
        pltpu.sync_copy(ids_hbm.at[pl.ds(g * N + base, BPW)],
                        idx_v.at[pl.ds(g * BPW, BPW)])
    for g in range(3):
        pltpu.async_copy(table_hbm.at[idx_v.at[pl.ds(g * BPW, BPW)]],
                         buf.at[pl.ds(g * BPW, BPW)], sem)
    for g in range(3):
        pltpu.make_async_copy(table_hbm.at[idx_v.at[pl.ds(g * BPW, BPW)]],
                              buf.at[pl.ds(g * BPW, BPW)], sem).wait()

    def shuf(q, _):
        for u in range(9):
            rv = cst[pl.ds(u * 16, 16)]
            cv = cst[pl.ds(144 + u * 16, 16)]
            vals = plsc.load_gather(buf, [8 * q + rv, cv])
            out_v[pl.ds(144 * q + 16 * u, 16)] = vals
        return 0

    lax.fori_loop(0, NBLK, shuf, 0)
    pltpu.sync_copy(out_v, out_hbm.at[pl.ds(wid * OUT_PW, OUT_PW)])


_mesh = plsc.VectorSubcoreMesh(core_axis_name="c", subcore_axis_name="s")

_sampler = functools.partial(
    pl.kernel,
    out_type=jax.ShapeDtypeStruct((N * 18,), jnp.float32),
    mesh=_mesh,
    scratch_types=[
        pltpu.VMEM((3 * BPW,), jnp.int32),
        pltpu.VMEM((3 * BPW, N_JOINTS * 3), jnp.float32),
        pltpu.VMEM((OUT_PW,), jnp.float32),
        pltpu.VMEM((288,), jnp.int32),
        pltpu.SemaphoreType.DMA,
    ],
    compiler_params=pltpu.CompilerParams(use_tc_tiling_on_sc=False,
                                         needs_layout_passes=False),
)(_sampler_body)


def _make_cst() -> np.ndarray:
    p = np.arange(144)
    n = p // 18
    j = (p % 18) // 3
    k = p % 3
    g = j % 3
    rv = 512 * g + n
    cv = 36 + 3 * j + k
    return np.concatenate([rv, cv]).astype(np.int32)


_CST = _make_cst()


@jax.jit
def kernel(euler_sample, rand_id1, rand_id2, rand_id3):
    table = euler_sample.reshape(DATASET_LEN, N_JOINTS * 3)
    ids = jnp.concatenate([rand_id1.astype(jnp.int32),
                           rand_id2.astype(jnp.int32),
                           rand_id3.astype(jnp.int32)])
    out = _sampler(table, ids, jnp.asarray(_CST))
    return out.reshape(N, 6, 3)


# speedup vs baseline: 1.0520x; 1.0520x over previous
import functools

import jax
import jax.numpy as jnp
import numpy as np
from jax import lax
from jax.experimental import pallas as pl
from jax.experimental.pallas import tpu as pltpu
from jax.experimental.pallas import tpu_sc as plsc

DATASET_LEN = 100000
ROW = 128
N = 16384
NUM_WORKERS = 32
BPW = N // NUM_WORKERS
HALF = BPW // 2
OUT_PH = HALF * 18
NBLK = OUT_PH // 144


def _sampler_body(table_hbm, ids_hbm, cst_hbm, out_hbm,
                  idx_v, buf, out_v, cst, sem):
    wid = lax.axis_index("s") * 2 + lax.axis_index("c")
    base = wid * BPW

    pltpu.sync_copy(cst_hbm, cst)
    for g in range(3):
        pltpu.sync_copy(ids_hbm.at[pl.ds(g * N + base, BPW)],
                        idx_v.at[pl.ds(g * BPW, BPW)])

    for h in range(2):
        for g in range(3):
            pltpu.async_copy(
                table_hbm.at[idx_v.at[pl.ds(g * BPW + h * HALF, HALF)]],
                buf.at[pl.ds(g * HALF, HALF)], sem)
        for g in range(3):
            pltpu.make_async_copy(
                table_hbm.at[idx_v.at[pl.ds(g * BPW + h * HALF, HALF)]],
                buf.at[pl.ds(g * HALF, HALF)], sem).wait()

        def shuf(q, _):
            for u in range(9):
                rv = cst[pl.ds(u * 16, 16)]
                cv = cst[pl.ds(144 + u * 16, 16)]
                vals = plsc.load_gather(buf, [8 * q + rv, cv])
                out_v[pl.ds(h * OUT_PH + 144 * q + 16 * u, 16)] = vals
            return 0

        lax.fori_loop(0, NBLK, shuf, 0)

    pltpu.sync_copy(out_v, out_hbm.at[pl.ds(wid * 2 * OUT_PH, 2 * OUT_PH)])


_mesh = plsc.VectorSubcoreMesh(core_axis_name="c", subcore_axis_name="s")

_sampler = functools.partial(
    pl.kernel,
    out_type=jax.ShapeDtypeStruct((N * 18,), jnp.float32),
    mesh=_mesh,
    scratch_types=[
        pltpu.VMEM((3 * BPW,), jnp.int32),
        pltpu.VMEM((3 * HALF, ROW), jnp.float32),
        pltpu.VMEM((2 * OUT_PH,), jnp.float32),
        pltpu.VMEM((288,), jnp.int32),
        pltpu.SemaphoreType.DMA,
    ],
    compiler_params=pltpu.CompilerParams(use_tc_tiling_on_sc=True,
                                         needs_layout_passes=False),
)(_sampler_body)


def _make_cst() -> np.ndarray:
    p = np.arange(144)
    n = p // 18
    j = (p % 18) // 3
    k = p % 3
    g = j % 3
    rv = HALF * g + n
    cv = 36 + 3 * j + k
    return np.concatenate([rv, cv]).astype(np.int32)


_CST = _make_cst()


@jax.jit
def kernel(euler_sample, rand_id1, rand_id2, rand_id3):
    table = euler_sample.reshape(DATASET_LEN, 72)
    table = jnp.concatenate(
        [table, jnp.zeros((DATASET_LEN, ROW - 72), jnp.float32)], axis=1)
    ids = jnp.concatenate([rand_id1.astype(jnp.int32),
                           rand_id2.astype(jnp.int32),
                           rand_id3.astype(jnp.int32)])
    out = _sampler(table, ids, jnp.asarray(_CST))
    return out.reshape(N, 6, 3)


# speedup vs baseline: 6.9457x; 6.6026x over previous
import functools

import jax
import jax.numpy as jnp
from jax import lax
from jax.experimental import pallas as pl
from jax.experimental.pallas import tpu as pltpu
from jax.experimental.pallas import tpu_sc as plsc

DATASET_LEN = 100000
IDT = 782
KSTRIDE = IDT * 128
N = 16384
NUM_WORKERS = 32
BPW = N // NUM_WORKERS
PAIRS = 3 * BPW
ROWS_PW = 6 * PAIRS
CHUNK = 128
NCHUNK = ROWS_PW // CHUNK


def _sampler_body(table_hbm, id1_hbm, id2_hbm, id3_hbm, out_hbm,
                  idx_v, gidx, colarr, buf, out_v, sem):
    wid = lax.axis_index("s") * 2 + lax.axis_index("c")
    base = wid * BPW

    pltpu.sync_copy(id1_hbm.at[pl.ds(base, BPW)], idx_v.at[pl.ds(0, BPW)])
    pltpu.sync_copy(id2_hbm.at[pl.ds(base, BPW)], idx_v.at[pl.ds(BPW, BPW)])
    pltpu.sync_copy(id3_hbm.at[pl.ds(base, BPW)], idx_v.at[pl.ds(2 * BPW, BPW)])

    lanes = lax.iota(jnp.int32, 16)

    def build(i, _):
        for g in range(3):
            ids = idx_v[pl.ds(g * BPW + i * 16, 16)]
            b = (ids & -128) + ((ids >> 3) & 15)
            pos0 = 288 * i + 18 * lanes + 6 * g
            plsc.store_scatter(colarr, [48 * i + 3 * lanes + g], ids & 7)
            for m in range(6):
                jpos = m & 1
                k = m >> 1
                c = k * KSTRIDE + g * 16 + jpos * 48
                plsc.store_scatter(gidx, [pos0 + m], b + c)
        return 0

    lax.fori_loop(0, BPW // 16, build, 0)

    def fire(c, _):
        pltpu.async_copy(table_hbm.at[gidx.at[pl.ds(c * CHUNK, CHUNK)]],
                         buf.at[pl.ds(c * CHUNK, CHUNK)], sem)
        return 0

    lax.fori_loop(0, NCHUNK, fire, 0)

    def drain(c, _):
        pltpu.make_async_copy(table_hbm.at[gidx.at[pl.ds(c * CHUNK, CHUNK)]],
                              buf.at[pl.ds(c * CHUNK, CHUNK)], sem).wait()
        return 0

    lax.fori_loop(0, NCHUNK, drain, 0)

    for j in range(6):
        g = j % 3
        jpos = j // 3
        for k in range(3):
            c = 6 * g + 2 * k + jpos

            def shuf(t, _, g=g, c=c, j=j, k=k):
                ntl = t >> 3
                v = t & 7
                n0 = 128 * ntl + 16 * v
                rowv = 18 * n0 + 18 * lanes + c
                colv = plsc.load_gather(colarr, [3 * n0 + 3 * lanes + g])
                vals = plsc.load_gather(buf, [rowv, colv])
                out_v[j, ntl, pl.ds(k * 128 + 16 * v, 16)] = vals
                return 0

            lax.fori_loop(0, 32, shuf, 0)

    for j in range(6):
        pltpu.sync_copy(out_v.at[j], out_hbm.at[j, pl.ds(4 * wid, 4)])


_mesh = plsc.VectorSubcoreMesh(core_axis_name="c", subcore_axis_name="s")

_sampler = functools.partial(
    pl.kernel,
    out_type=jax.ShapeDtypeStruct((6, 128, 512), jnp.float32),
    mesh=_mesh,
    scratch_types=[
        pltpu.VMEM((PAIRS,), jnp.int32),
        pltpu.VMEM((ROWS_PW,), jnp.int32),
        pltpu.VMEM((PAIRS,), jnp.int32),
        pltpu.VMEM((ROWS_PW, 8), jnp.float32),
        pltpu.VMEM((6, 4, 512), jnp.float32),
        pltpu.SemaphoreType.DMA,
    ],
    compiler_params=pltpu.CompilerParams(use_tc_tiling_on_sc=False,
                                         needs_layout_passes=False),
)(_sampler_body)


@jax.jit
def kernel(euler_sample, rand_id1, rand_id2, rand_id3):
    s = euler_sample[:, 12:18, :]
    sp = jnp.pad(s, ((0, 96), (0, 2), (0, 0)))
    t = jnp.transpose(sp, (2, 1, 0))
    r = t.reshape(3, 8, IDT, 128)
    table = jnp.transpose(r, (0, 2, 1, 3)).reshape(3 * IDT * 8 * 16, 8)
    out = _sampler(table,
                   rand_id1.astype(jnp.int32),
                   rand_id2.astype(jnp.int32),
                   rand_id3.astype(jnp.int32))
    o = out.reshape(6, 128, 4, 128)
    o = jnp.transpose(o, (1, 3, 0, 2)).reshape(N, 6, 4)
    return o[:, :, :3]


# speedup vs baseline: 7.7525x; 1.1162x over previous
import functools

import jax
import jax.numpy as jnp
from jax import lax
from jax.experimental import pallas as pl
from jax.experimental.pallas import tpu as pltpu
from jax.experimental.pallas import tpu_sc as plsc

DATASET_LEN = 100000
IDT = 782
KSTRIDE = IDT * 128
N = 16384
NUM_WORKERS = 32
BPW = N // NUM_WORKERS
PAIRS = 3 * BPW
ROWS_PW = 6 * PAIRS
CHUNK = 128
NCHUNK = ROWS_PW // CHUNK


def _sampler_body(table_hbm, id1_hbm, id2_hbm, id3_hbm, out_hbm,
                  idx_v, gidx, colarr, buf, out_v, sem):
    wid = lax.axis_index("s") * 2 + lax.axis_index("c")
    base = wid * BPW

    pltpu.sync_copy(id1_hbm.at[pl.ds(base, BPW)], idx_v.at[pl.ds(0, BPW)])
    pltpu.sync_copy(id2_hbm.at[pl.ds(base, BPW)], idx_v.at[pl.ds(BPW, BPW)])
    pltpu.sync_copy(id3_hbm.at[pl.ds(base, BPW)], idx_v.at[pl.ds(2 * BPW, BPW)])

    lanes = lax.iota(jnp.int32, 16)

    def build(i, _):
        for g in range(3):
            ids = idx_v[pl.ds(g * BPW + i * 16, 16)]
            b = (ids & -128) + ((ids >> 3) & 15)
            pos0 = 288 * i + 18 * lanes + 6 * g
            plsc.store_scatter(colarr, [48 * i + 3 * lanes + g], ids & 7)
            for m in range(6):
                jpos = m & 1
                k = m >> 1
                joff = 4 + g + 3 * jpos
                jt, jj = joff >> 3, joff & 7
                c = k * 2 * KSTRIDE + jt * KSTRIDE + jj * 16
                plsc.store_scatter(gidx, [pos0 + m], b + c)
        return 0

    lax.fori_loop(0, BPW // 16, build, 0)

    def fire(c, _):
        pltpu.async_copy(table_hbm.at[gidx.at[pl.ds(c * CHUNK, CHUNK)]],
                         buf.at[pl.ds(c * CHUNK, CHUNK)], sem)
        return 0

    lax.fori_loop(0, NCHUNK, fire, 0)

    def drain(c, _):
        pltpu.make_async_copy(table_hbm.at[gidx.at[pl.ds(c * CHUNK, CHUNK)]],
                              buf.at[pl.ds(c * CHUNK, CHUNK)], sem).wait()
        return 0

    lax.fori_loop(0, NCHUNK, drain, 0)

    for j in range(6):
        g = j % 3
        jpos = j // 3
        for k in range(3):
            c = 6 * g + 2 * k + jpos

            def shuf(t, _, g=g, c=c, j=j, k=k):
                ntl = t >> 3
                v = t & 7
                n0 = 128 * ntl + 16 * v
                rowv = 18 * n0 + 18 * lanes + c
                colv = plsc.load_gather(colarr, [3 * n0 + 3 * lanes + g])
                vals = plsc.load_gather(buf, [rowv, colv])
                out_v[j, ntl, pl.ds(k * 128 + 16 * v, 16)] = vals
                return 0

            lax.fori_loop(0, 32, shuf, 0)

    for j in range(6):
        pltpu.sync_copy(out_v.at[j], out_hbm.at[j, pl.ds(4 * wid, 4)])


_mesh = plsc.VectorSubcoreMesh(core_axis_name="c", subcore_axis_name="s")

_sampler = functools.partial(
    pl.kernel,
    out_type=jax.ShapeDtypeStruct((6, 128, 512), jnp.float32),
    mesh=_mesh,
    scratch_types=[
        pltpu.VMEM((PAIRS,), jnp.int32),
        pltpu.VMEM((ROWS_PW,), jnp.int32),
        pltpu.VMEM((PAIRS,), jnp.int32),
        pltpu.VMEM((ROWS_PW, 8), jnp.float32),
        pltpu.VMEM((6, 4, 512), jnp.float32),
        pltpu.SemaphoreType.DMA,
    ],
    compiler_params=pltpu.CompilerParams(use_tc_tiling_on_sc=False,
                                         needs_layout_passes=False),
)(_sampler_body)


@jax.jit
def kernel(euler_sample, rand_id1, rand_id2, rand_id3):
    s = euler_sample[:, 8:24, :]
    sp = jax.lax.dynamic_update_slice(
        jnp.zeros((100096, 16, 3), jnp.float32), s, (0, 0, 0))
    t = jnp.transpose(sp, (2, 1, 0))
    r = t.reshape(3, 2, 8, IDT, 128)
    table = jnp.transpose(r, (0, 1, 3, 2, 4)).reshape(3 * 2 * IDT * 8 * 16,
                                                      8)
    out = _sampler(table,
                   rand_id1.astype(jnp.int32),
                   rand_id2.astype(jnp.int32),
                   rand_id3.astype(jnp.int32))
    o = out.reshape(6, 128, 4, 128)
    o = jnp.transpose(o, (1, 3, 0, 2)).reshape(N, 6, 4)
    return o[:, :, :3]


# speedup vs baseline: 8.4812x; 1.0940x over previous
import functools

import jax
import jax.numpy as jnp
from jax import lax
from jax.experimental import pallas as pl
from jax.experimental.pallas import tpu as pltpu
from jax.experimental.pallas import tpu_sc as plsc

DATASET_LEN = 100000
IDT = 782
KSTRIDE = IDT * 128
N = 16384
NUM_WORKERS = 32
BPW = N // NUM_WORKERS
PAIRS = 3 * BPW
ROWS_PW = 6 * PAIRS
CHUNK = 128
NCHUNK = ROWS_PW // CHUNK


def _sampler_body(table_hbm, id1_hbm, id2_hbm, id3_hbm, out_hbm,
                  idx_v, gidx, buf, out_v, sem):
    wid = lax.axis_index("s") * 2 + lax.axis_index("c")
    base = wid * BPW

    pltpu.sync_copy(id1_hbm.at[pl.ds(base, BPW)], idx_v.at[pl.ds(0, BPW)])
    pltpu.sync_copy(id2_hbm.at[pl.ds(base, BPW)], idx_v.at[pl.ds(BPW, BPW)])
    pltpu.sync_copy(id3_hbm.at[pl.ds(base, BPW)], idx_v.at[pl.ds(2 * BPW, BPW)])

    lanes = lax.iota(jnp.int32, 16)

    def _q(g, m):
        return (g + 3 * (m & 1)) * 3 + (m >> 1)

    def build(i, _):
        for g in range(3):
            ids = idx_v[pl.ds(g * BPW + i * 16, 16)]
            b = (ids & -128) + ((ids >> 3) & 15)
            for m in range(6):
                jpos = m & 1
                k = m >> 1
                joff = 4 + g + 3 * jpos
                jt, jj = joff >> 3, joff & 7
                c = k * 2 * KSTRIDE + jt * KSTRIDE + jj * 16
                gidx[pl.ds(_q(g, m) * BPW + i * 16, 16)] = b + c
        return 0

    lax.fori_loop(0, BPW // 16, build, 0)

    def fire(c, _):
        pltpu.async_copy(table_hbm.at[gidx.at[pl.ds(c * CHUNK, CHUNK)]],
                         buf.at[pl.ds(c * CHUNK, CHUNK)], sem)
        return 0

    lax.fori_loop(0, NCHUNK, fire, 0)

    for j in range(6):
        g = j % 3
        for k in range(3):
            q = j * 3 + k

            def drain(c, _, q=q):
                s = q * 4 + c
                pltpu.make_async_copy(
                    table_hbm.at[gidx.at[pl.ds(s * CHUNK, CHUNK)]],
                    buf.at[pl.ds(s * CHUNK, CHUNK)], sem).wait()
                return 0

            lax.fori_loop(0, 4, drain, 0)

            def shuf(t, _, g=g, q=q, j=j, k=k):
                n0 = 16 * t
                ids = idx_v[pl.ds(g * BPW + n0, 16)]
                rowv = q * BPW + n0 + lanes
                vals = plsc.load_gather(buf, [rowv, ids & 7])
                out_v[j, t >> 3, pl.ds(k * 128 + 16 * (t & 7), 16)] = vals
                return 0

            lax.fori_loop(0, 32, shuf, 0)

    for j in range(6):
        pltpu.sync_copy(out_v.at[j], out_hbm.at[j, pl.ds(4 * wid, 4)])


_mesh = plsc.VectorSubcoreMesh(core_axis_name="c", subcore_axis_name="s")

_sampler = functools.partial(
    pl.kernel,
    out_type=jax.ShapeDtypeStruct((6, 128, 512), jnp.float32),
    mesh=_mesh,
    scratch_types=[
        pltpu.VMEM((PAIRS,), jnp.int32),
        pltpu.VMEM((ROWS_PW,), jnp.int32),
        pltpu.VMEM((ROWS_PW, 8), jnp.float32),
        pltpu.VMEM((6, 4, 512), jnp.float32),
        pltpu.SemaphoreType.DMA,
    ],
    compiler_params=pltpu.CompilerParams(use_tc_tiling_on_sc=False,
                                         needs_layout_passes=False),
)(_sampler_body)


@jax.jit
def kernel(euler_sample, rand_id1, rand_id2, rand_id3):
    s = euler_sample[:, 8:24, :]
    sp = jax.lax.dynamic_update_slice(
        jnp.zeros((100096, 16, 3), jnp.float32), s, (0, 0, 0))
    t = jnp.transpose(sp, (2, 1, 0))
    r = t.reshape(3, 2, 8, IDT, 128)
    table = jnp.transpose(r, (0, 1, 3, 2, 4)).reshape(3 * 2 * IDT * 8 * 16,
                                                      8)
    out = _sampler(table,
                   rand_id1.astype(jnp.int32),
                   rand_id2.astype(jnp.int32),
                   rand_id3.astype(jnp.int32))
    o = out.reshape(6, 128, 4, 128)
    o = jnp.transpose(o, (1, 3, 0, 2)).reshape(N, 6, 4)
    return o[:, :, :3]


# speedup vs baseline: 8.4843x; 1.0004x over previous
import functools

import jax
import jax.numpy as jnp
from jax import lax
from jax.experimental import pallas as pl
from jax.experimental.pallas import tpu as pltpu
from jax.experimental.pallas import tpu_sc as plsc

DATASET_LEN = 100000
IDT = 782
KSTRIDE = IDT * 128
N = 16384
NUM_WORKERS = 32
BPW = N // NUM_WORKERS
PAIRS = 3 * BPW
ROWS_PW = 6 * PAIRS
CHUNK = 512
NCHUNK = ROWS_PW // CHUNK


def _sampler_body(table_hbm, id1_hbm, id2_hbm, id3_hbm, out_hbm,
                  idx_v, gidx, buf, out_v, sem):
    wid = lax.axis_index("s") * 2 + lax.axis_index("c")
    base = wid * BPW

    pltpu.sync_copy(id1_hbm.at[pl.ds(base, BPW)], idx_v.at[pl.ds(0, BPW)])
    pltpu.sync_copy(id2_hbm.at[pl.ds(base, BPW)], idx_v.at[pl.ds(BPW, BPW)])
    pltpu.sync_copy(id3_hbm.at[pl.ds(base, BPW)], idx_v.at[pl.ds(2 * BPW, BPW)])

    lanes = lax.iota(jnp.int32, 16)

    def _q(g, m):
        return (g + 3 * (m & 1)) * 3 + (m >> 1)

    def build(i, _):
        for g in range(3):
            ids = idx_v[pl.ds(g * BPW + i * 16, 16)]
            b = (ids & -128) + ((ids >> 3) & 15)
            for m in range(6):
                jpos = m & 1
                k = m >> 1
                joff = 4 + g + 3 * jpos
                jt, jj = joff >> 3, joff & 7
                c = k * 2 * KSTRIDE + jt * KSTRIDE + jj * 16
                gidx[pl.ds(_q(g, m) * BPW + i * 16, 16)] = b + c
        return 0

    lax.fori_loop(0, BPW // 16, build, 0)

    def fire(c, _):
        pltpu.async_copy(table_hbm.at[gidx.at[pl.ds(c * CHUNK, CHUNK)]],
                         buf.at[pl.ds(c * CHUNK, CHUNK)], sem)
        return 0

    lax.fori_loop(0, NCHUNK, fire, 0)

    for j in range(6):
        g = j % 3
        for k in range(3):
            q = j * 3 + k

            pltpu.make_async_copy(
                table_hbm.at[gidx.at[pl.ds(q * CHUNK, CHUNK)]],
                buf.at[pl.ds(q * CHUNK, CHUNK)], sem).wait()

            def shuf(t, _, g=g, q=q, j=j, k=k):
                n0 = 16 * t
                ids = idx_v[pl.ds(g * BPW + n0, 16)]
                rowv = q * BPW + n0 + lanes
                vals = plsc.load_gather(buf, [rowv, ids & 7])
                out_v[j, t >> 3, pl.ds(k * 128 + 16 * (t & 7), 16)] = vals
                return 0

            lax.fori_loop(0, 32, shuf, 0)

    for j in range(6):
        pltpu.sync_copy(out_v.at[j], out_hbm.at[j, pl.ds(4 * wid, 4)])


_mesh = plsc.VectorSubcoreMesh(core_axis_name="c", subcore_axis_name="s")

_sampler = functools.partial(
    pl.kernel,
    out_type=jax.ShapeDtypeStruct((6, 128, 512), jnp.float32),
    mesh=_mesh,
    scratch_types=[
        pltpu.VMEM((PAIRS,), jnp.int32),
        pltpu.VMEM((ROWS_PW,), jnp.int32),
        pltpu.VMEM((ROWS_PW, 8), jnp.float32),
        pltpu.VMEM((6, 4, 512), jnp.float32),
        pltpu.SemaphoreType.DMA,
    ],
    compiler_params=pltpu.CompilerParams(use_tc_tiling_on_sc=False,
                                         needs_layout_passes=False),
)(_sampler_body)


@jax.jit
def kernel(euler_sample, rand_id1, rand_id2, rand_id3):
    s = euler_sample[:, 8:24, :]
    sp = jax.lax.dynamic_update_slice(
        jnp.zeros((100096, 16, 3), jnp.float32), s, (0, 0, 0))
    t = jnp.transpose(sp, (2, 1, 0))
    r = t.reshape(3, 2, 8, IDT, 128)
    table = jnp.transpose(r, (0, 1, 3, 2, 4)).reshape(3 * 2 * IDT * 8 * 16,
                                                      8)
    out = _sampler(table,
                   rand_id1.astype(jnp.int32),
                   rand_id2.astype(jnp.int32),
                   rand_id3.astype(jnp.int32))
    o = out.reshape(6, 128, 4, 128)
    o = jnp.transpose(o, (1, 3, 0, 2)).reshape(N, 6, 4)
    return o[:, :, :3]
